# baseline (speedup 1.0000x reference)
import functools

import jax
import jax.numpy as jnp
from jax import lax
from jax.experimental import pallas as pl
from jax.experimental.pallas import tpu as pltpu
from jax.experimental.pallas import tpu_sc as plsc

N = 10000
E = 160000
D = 256
NC = 2
NS = 16
NW = NC * NS
N_PAD = 10240
E_PAD = 163840
ECH = 512
E_PER_W = E_PAD // NW
E_CHUNKS = E_PER_W // ECH
XCH = 64
X_PER_W = N_PAD // NW
X_CHUNKS = X_PER_W // XCH
RPS = N_PAD // NS


def _mesh():
    return plsc.VectorSubcoreMesh(core_axis_name="c", subcore_axis_name="s")


@functools.partial(
    pl.kernel,
    out_type=jax.ShapeDtypeStruct((N_PAD, D), jnp.float32),
    mesh=_mesh(),
    scratch_types=[
        pltpu.VMEM((X_CHUNKS, XCH), jnp.int32),
        pltpu.VMEM((XCH, D), jnp.float32),
        pltpu.VMEM((XCH, D), jnp.float32),
        pltpu.SemaphoreType.DMA,
        pltpu.SemaphoreType.DMA,
    ],
    name="sc_embed",
)
def _sc_embed(x2d, table, h0, xidx, rows_a, rows_b, sem_a, sem_b):
    cid = lax.axis_index("c")
    sid = lax.axis_index("s")
    wid = sid * NC + cid
    pltpu.sync_copy(x2d.at[wid], xidx)
    bufs = (rows_a, rows_b)
    sems = (sem_a, sem_b)
    descs = [None, None]
    for j in range(X_CHUNKS):
        b = j % 2
        if descs[b] is not None:
            descs[b].wait()
            pltpu.sync_copy(bufs[b],
                            h0.at[pl.ds(wid * X_PER_W + (j - 2) * XCH, XCH)])
        descs[b] = pltpu.async_copy(table.at[xidx.at[j]], bufs[b], sems[b])
    for j in range(X_CHUNKS - 2, X_CHUNKS):
        b = j % 2
        descs[b].wait()
        pltpu.sync_copy(bufs[b],
                        h0.at[pl.ds(wid * X_PER_W + j * XCH, XCH)])


@functools.partial(
    pl.kernel,
    out_type=jax.ShapeDtypeStruct((NC, N_PAD, 16), jnp.float32),
    mesh=_mesh(),
    scratch_types=[
        pltpu.VMEM((E_CHUNKS, ECH), jnp.int32),
        pltpu.VMEM((ECH, 16), jnp.float32),
        pltpu.VMEM_SHARED((N_PAD, 16), jnp.float32),
        pltpu.SemaphoreType.DMA,
    ],
    compiler_params=pltpu.CompilerParams(use_tc_tiling_on_sc=False),
    name="sc_deg",
)
def _sc_deg(col2d, zcol, ones, degp, cidx, ones_v, deg_sh, sem):
    cid = lax.axis_index("c")
    sid = lax.axis_index("s")
    wid = sid * NC + cid
    pltpu.sync_copy(zcol.at[pl.ds(sid * RPS, RPS)],
                    deg_sh.at[pl.ds(sid * RPS, RPS)])
    pltpu.sync_copy(ones, ones_v)
    pltpu.sync_copy(col2d.at[wid], cidx)
    plsc.subcore_barrier()

    for k in range(E_CHUNKS):
        pltpu.async_copy(ones_v, deg_sh.at[cidx.at[k]], sem, add=True)
    for _ in range(E_CHUNKS):
        pltpu.make_async_copy(ones_v, deg_sh.at[cidx.at[0]], sem).wait()
    plsc.subcore_barrier()
    pltpu.sync_copy(deg_sh.at[pl.ds(sid * RPS, RPS)],
                    degp.at[cid, pl.ds(sid * RPS, RPS)])


NCH_TOT = E_PAD // ECH
C_FAST = 16
C_SLOW = NCH_TOT // NS - C_FAST


def _make_agg(dm):
    @functools.partial(
        pl.kernel,
        out_type=jax.ShapeDtypeStruct((NC, N_PAD, dm), jnp.float32),
        mesh=_mesh(),
        scratch_types=[
            pltpu.VMEM((C_FAST, ECH), jnp.int32),
            pltpu.VMEM((C_FAST, ECH), jnp.int32),
            [pltpu.VMEM((ECH, dm), jnp.float32)] * 2,
            pltpu.VMEM_SHARED((N_PAD, dm), jnp.float32),
            [pltpu.SemaphoreType.DMA] * 2,
        ],
        compiler_params=pltpu.CompilerParams(use_tc_tiling_on_sc=False),
        name=f"sc_agg{dm}",
    )
    def agg(row2d, col2d, s_hbm, zeros_hbm, aggp,
            ridx, cidx, msgs, agg_sh, sem_g):
        cid = lax.axis_index("c")
        sid = lax.axis_index("s")
        pltpu.sync_copy(zeros_hbm.at[pl.ds(sid * RPS, RPS)],
                        agg_sh.at[pl.ds(sid * RPS, RPS)])

        @pl.when(cid == 1)
        def _():
            start = sid * C_FAST
            pltpu.sync_copy(row2d.at[pl.ds(start, C_FAST)], ridx)
            pltpu.sync_copy(col2d.at[pl.ds(start, C_FAST)], cidx)

        @pl.when(cid == 0)
        def _():
            start = NS * C_FAST + sid * C_SLOW
            pltpu.sync_copy(row2d.at[pl.ds(start, C_SLOW)],
                            ridx.at[pl.ds(0, C_SLOW)])
            pltpu.sync_copy(col2d.at[pl.ds(start, C_SLOW)],
                            cidx.at[pl.ds(0, C_SLOW)])

        nch = jnp.where(cid == 1, C_FAST, C_SLOW)
        plsc.subcore_barrier()

        pltpu.async_copy(s_hbm.at[ridx.at[0]], msgs[0], sem_g[0])

        def pair(j2, c):
            ja = 2 * j2
            jb = ja + 1
            ja_next = jnp.minimum(ja + 2, nch - 1)
            pltpu.async_copy(s_hbm.at[ridx.at[jb]], msgs[1], sem_g[1])
            pltpu.make_async_copy(s_hbm.at[ridx.at[0]], msgs[0],
                                  sem_g[0]).wait()
            pltpu.sync_copy(msgs[0], agg_sh.at[cidx.at[ja]], add=True)
            pltpu.async_copy(s_hbm.at[ridx.at[ja_next]], msgs[0], sem_g[0])
            pltpu.make_async_copy(s_hbm.at[ridx.at[0]], msgs[1],
                                  sem_g[1]).wait()
            pltpu.sync_copy(msgs[1], agg_sh.at[cidx.at[jb]], add=True)
            return c

        lax.fori_loop(0, nch // 2, pair, 0)
        pltpu.make_async_copy(s_hbm.at[ridx.at[0]], msgs[0], sem_g[0]).wait()
        plsc.subcore_barrier()
        pltpu.sync_copy(agg_sh.at[pl.ds(sid * RPS, RPS)],
                        aggp.at[cid, pl.ds(sid * RPS, RPS)])

    return agg


_agg64 = _make_agg(64)
_agg32 = _make_agg(32)


BR = 1024


def _tc1_body(h0_ref, degp_ref, w1_ref, s1_ref, dinv_ref):
    deg = degp_ref[0, :, 0:1] + degp_ref[1, :, 0:1] + 1.0
    dinv = lax.rsqrt(deg)
    xw = jnp.dot(h0_ref[...], w1_ref[...], preferred_element_type=jnp.float32)
    s1_ref[...] = xw * dinv
    dinv_ref[...] = dinv


_tc1 = pl.pallas_call(
    _tc1_body,
    grid=(N_PAD // BR,),
    in_specs=[
        pl.BlockSpec((BR, D), lambda i: (i, 0)),
        pl.BlockSpec((NC, BR, 16), lambda i: (0, i, 0)),
        pl.BlockSpec((D, 64), lambda i: (0, 0)),
    ],
    out_specs=[
        pl.BlockSpec((BR, 64), lambda i: (i, 0)),
        pl.BlockSpec((BR, 1), lambda i: (i, 0)),
    ],
    out_shape=[
        jax.ShapeDtypeStruct((N_PAD, 64), jnp.float32),
        jax.ShapeDtypeStruct((N_PAD, 1), jnp.float32),
    ],
)


def _tc2_body(s1_ref, aggp_ref, dinv_ref, b1_ref, w2_ref, s2_ref):
    dinv = dinv_ref[...]
    pre = (aggp_ref[0] + aggp_ref[1] + s1_ref[...]) * dinv + b1_ref[...]
    h1 = jnp.maximum(pre, 0.0)
    s2_ref[...] = jnp.dot(h1, w2_ref[...],
                          preferred_element_type=jnp.float32) * dinv


_tc2 = pl.pallas_call(
    _tc2_body,
    grid=(N_PAD // BR,),
    in_specs=[
        pl.BlockSpec((BR, 64), lambda i: (i, 0)),
        pl.BlockSpec((NC, BR, 64), lambda i: (0, i, 0)),
        pl.BlockSpec((BR, 1), lambda i: (i, 0)),
        pl.BlockSpec((1, 64), lambda i: (0, 0)),
        pl.BlockSpec((64, 32), lambda i: (0, 0)),
    ],
    out_specs=pl.BlockSpec((BR, 32), lambda i: (i, 0)),
    out_shape=jax.ShapeDtypeStruct((N_PAD, 32), jnp.float32),
)


def _tc3_body(s2_ref, aggp_ref, dinv_ref, b2_ref, wg_ref, bg_ref,
              wh_ref, bh_ref, y_ref):
    h2 = (aggp_ref[0] + aggp_ref[1] + s2_ref[...]) * dinv_ref[...] + b2_ref[...]
    gate = jnp.dot(h2, wg_ref[...], preferred_element_type=jnp.float32)
    gate = gate + bg_ref[...]
    gate = 1.0 / (1.0 + jnp.exp(-gate))
    valid = lax.broadcasted_iota(jnp.int32, (N_PAD, 1), 0) < N
    g = jnp.where(valid, gate, -1e30)
    m = jnp.max(g)
    e = jnp.where(valid, jnp.exp(g - m), 0.0)
    hg = jnp.sum(e * h2, axis=0, keepdims=True) / jnp.sum(e)
    y_ref[...] = jnp.dot(hg, wh_ref[...],
                         preferred_element_type=jnp.float32) + bh_ref[...]


_tc3 = pl.pallas_call(
    _tc3_body,
    in_specs=[
        pl.BlockSpec((N_PAD, 32), lambda: (0, 0)),
        pl.BlockSpec((NC, N_PAD, 32), lambda: (0, 0, 0)),
        pl.BlockSpec((N_PAD, 1), lambda: (0, 0)),
        pl.BlockSpec((1, 32), lambda: (0, 0)),
        pl.BlockSpec((32, 1), lambda: (0, 0)),
        pl.BlockSpec((1, 1), lambda: (0, 0)),
        pl.BlockSpec((32, 2), lambda: (0, 0)),
        pl.BlockSpec((1, 2), lambda: (0, 0)),
    ],
    out_specs=pl.BlockSpec((1, 2), lambda: (0, 0)),
    out_shape=jax.ShapeDtypeStruct((1, 2), jnp.float32),
)


def kernel(x, edge_index, edge_attr, embed_table, edge_embed_table,
           W1, b1, W2, b2, Wg, bg, Wh, bh):
    del edge_attr, edge_embed_table
    xi = x[:, 0].astype(jnp.int32)
    x_pad = jnp.concatenate(
        [xi, jnp.zeros((N_PAD - N,), jnp.int32)]).reshape(NW, X_CHUNKS, XCH)
    row = edge_index[0].astype(jnp.int32)
    col = edge_index[1].astype(jnp.int32)
    row_flat = jnp.concatenate([row, jnp.zeros((E_PAD - E,), jnp.int32)])
    col_flat = jnp.concatenate([col, jnp.full((E_PAD - E,), N, jnp.int32)])
    row_ch = row_flat.reshape(NCH_TOT, ECH)
    col_ch = col_flat.reshape(NCH_TOT, ECH)
    col_pad = col_flat.reshape(NW, E_CHUNKS, ECH)
    zcol = jnp.zeros((N_PAD, 16), jnp.float32)
    ones = jnp.ones((ECH, 16), jnp.float32)
    z64 = jnp.zeros((N_PAD, 64), jnp.float32)
    z32 = jnp.zeros((N_PAD, 32), jnp.float32)

    h0 = _sc_embed(x_pad, embed_table)
    degp = _sc_deg(col_pad, zcol, ones)
    s1, dinv = _tc1(h0, degp, W1)
    aggp1 = _agg64(row_ch, col_ch, s1, z64)
    s2 = _tc2(s1, aggp1, dinv, b1.reshape(1, 64), W2)
    aggp2 = _agg32(row_ch, col_ch, s2, z32)
    y = _tc3(s2, aggp2, dinv, b2.reshape(1, 32), Wg, bg.reshape(1, 1),
             Wh, bh.reshape(1, 2))
    return y


# speedup vs baseline: 1.0607x; 1.0607x over previous
import functools

import jax
import jax.numpy as jnp
from jax import lax
from jax.experimental import pallas as pl
from jax.experimental.pallas import tpu as pltpu
from jax.experimental.pallas import tpu_sc as plsc

N = 10000
E = 160000
D = 256
NC = 2
NS = 16
NW = NC * NS
N_PAD = 10240
E_PAD = 163840
ECH = 512
E_PER_W = E_PAD // NW
E_CHUNKS = E_PER_W // ECH
XCH = 64
X_PER_W = N_PAD // NW
X_CHUNKS = X_PER_W // XCH
RPS = N_PAD // NS


def _mesh():
    return plsc.VectorSubcoreMesh(core_axis_name="c", subcore_axis_name="s")


@functools.partial(
    pl.kernel,
    out_type=jax.ShapeDtypeStruct((N_PAD, D), jnp.float32),
    mesh=_mesh(),
    scratch_types=[
        pltpu.VMEM((X_CHUNKS, XCH), jnp.int32),
        pltpu.VMEM((XCH, D), jnp.float32),
        pltpu.VMEM((XCH, D), jnp.float32),
        pltpu.SemaphoreType.DMA,
        pltpu.SemaphoreType.DMA,
    ],
    name="sc_embed",
)
def _sc_embed(x2d, table, h0, xidx, rows_a, rows_b, sem_a, sem_b):
    cid = lax.axis_index("c")
    sid = lax.axis_index("s")
    wid = sid * NC + cid
    pltpu.sync_copy(x2d.at[wid], xidx)
    bufs = (rows_a, rows_b)
    sems = (sem_a, sem_b)
    descs = [None, None]
    for j in range(X_CHUNKS):
        b = j % 2
        if descs[b] is not None:
            descs[b].wait()
            pltpu.sync_copy(bufs[b],
                            h0.at[pl.ds(wid * X_PER_W + (j - 2) * XCH, XCH)])
        descs[b] = pltpu.async_copy(table.at[xidx.at[j]], bufs[b], sems[b])
    for j in range(X_CHUNKS - 2, X_CHUNKS):
        b = j % 2
        descs[b].wait()
        pltpu.sync_copy(bufs[b],
                        h0.at[pl.ds(wid * X_PER_W + j * XCH, XCH)])


@functools.partial(
    pl.kernel,
    out_type=jax.ShapeDtypeStruct((NC, N_PAD, 16), jnp.float32),
    mesh=_mesh(),
    scratch_types=[
        pltpu.VMEM((E_CHUNKS, ECH), jnp.int32),
        pltpu.VMEM((ECH, 16), jnp.float32),
        pltpu.VMEM_SHARED((N_PAD, 16), jnp.float32),
        pltpu.SemaphoreType.DMA,
    ],
    compiler_params=pltpu.CompilerParams(use_tc_tiling_on_sc=False),
    name="sc_deg",
)
def _sc_deg(col2d, zcol, ones, degp, cidx, ones_v, deg_sh, sem):
    cid = lax.axis_index("c")
    sid = lax.axis_index("s")
    wid = sid * NC + cid
    pltpu.sync_copy(zcol.at[pl.ds(sid * RPS, RPS)],
                    deg_sh.at[pl.ds(sid * RPS, RPS)])
    pltpu.sync_copy(ones, ones_v)
    pltpu.sync_copy(col2d.at[wid], cidx)
    plsc.subcore_barrier()

    for k in range(E_CHUNKS):
        pltpu.async_copy(ones_v, deg_sh.at[cidx.at[k]], sem, add=True)
    for _ in range(E_CHUNKS):
        pltpu.make_async_copy(ones_v, deg_sh.at[cidx.at[0]], sem).wait()
    plsc.subcore_barrier()
    pltpu.sync_copy(deg_sh.at[pl.ds(sid * RPS, RPS)],
                    degp.at[cid, pl.ds(sid * RPS, RPS)])


def _make_agg(dm):
    @functools.partial(
        pl.kernel,
        out_type=jax.ShapeDtypeStruct((NC, N_PAD, dm), jnp.float32),
        mesh=_mesh(),
        scratch_types=[
            pltpu.VMEM((E_CHUNKS, ECH), jnp.int32),
            pltpu.VMEM((E_CHUNKS, ECH), jnp.int32),
            [pltpu.VMEM((ECH, dm), jnp.float32)] * 2,
            pltpu.VMEM_SHARED((N_PAD, dm), jnp.float32),
            [pltpu.SemaphoreType.DMA] * 2,
        ],
        compiler_params=pltpu.CompilerParams(use_tc_tiling_on_sc=False),
        name=f"sc_agg{dm}",
    )
    def agg(row2d, col2d, s_hbm, zeros_hbm, aggp,
            ridx, cidx, msgs, agg_sh, sem_g):
        cid = lax.axis_index("c")
        sid = lax.axis_index("s")
        wid = sid * NC + cid
        nch = E_CHUNKS
        pltpu.sync_copy(zeros_hbm.at[pl.ds(sid * RPS, RPS)],
                        agg_sh.at[pl.ds(sid * RPS, RPS)])
        pltpu.sync_copy(row2d.at[wid], ridx)
        pltpu.sync_copy(col2d.at[wid], cidx)
        plsc.subcore_barrier()

        pltpu.async_copy(s_hbm.at[ridx.at[0]], msgs[0], sem_g[0])

        def pair(j2, c):
            ja = 2 * j2
            jb = ja + 1
            ja_next = jnp.minimum(ja + 2, nch - 1)
            pltpu.async_copy(s_hbm.at[ridx.at[jb]], msgs[1], sem_g[1])
            pltpu.make_async_copy(s_hbm.at[ridx.at[0]], msgs[0],
                                  sem_g[0]).wait()
            pltpu.sync_copy(msgs[0], agg_sh.at[cidx.at[ja]], add=True)
            pltpu.async_copy(s_hbm.at[ridx.at[ja_next]], msgs[0], sem_g[0])
            pltpu.make_async_copy(s_hbm.at[ridx.at[0]], msgs[1],
                                  sem_g[1]).wait()
            pltpu.sync_copy(msgs[1], agg_sh.at[cidx.at[jb]], add=True)
            return c

        lax.fori_loop(0, nch // 2, pair, 0)
        pltpu.make_async_copy(s_hbm.at[ridx.at[0]], msgs[0], sem_g[0]).wait()
        plsc.subcore_barrier()
        pltpu.sync_copy(agg_sh.at[pl.ds(sid * RPS, RPS)],
                        aggp.at[cid, pl.ds(sid * RPS, RPS)])

    return agg


_agg64 = _make_agg(64)
_agg32 = _make_agg(32)


BR = 1024


def _tc1_body(h0_ref, degp_ref, w1_ref, s1_ref, dinv_ref):
    deg = degp_ref[0, :, 0:1] + degp_ref[1, :, 0:1] + 1.0
    dinv = lax.rsqrt(deg)
    xw = jnp.dot(h0_ref[...], w1_ref[...], preferred_element_type=jnp.float32)
    s1_ref[...] = xw * dinv
    dinv_ref[...] = dinv


_tc1 = pl.pallas_call(
    _tc1_body,
    grid=(N_PAD // BR,),
    in_specs=[
        pl.BlockSpec((BR, D), lambda i: (i, 0)),
        pl.BlockSpec((NC, BR, 16), lambda i: (0, i, 0)),
        pl.BlockSpec((D, 64), lambda i: (0, 0)),
    ],
    out_specs=[
        pl.BlockSpec((BR, 64), lambda i: (i, 0)),
        pl.BlockSpec((BR, 1), lambda i: (i, 0)),
    ],
    out_shape=[
        jax.ShapeDtypeStruct((N_PAD, 64), jnp.float32),
        jax.ShapeDtypeStruct((N_PAD, 1), jnp.float32),
    ],
)


def _tc2_body(s1_ref, aggp_ref, dinv_ref, b1_ref, w2_ref, s2_ref):
    dinv = dinv_ref[...]
    pre = (aggp_ref[0] + aggp_ref[1] + s1_ref[...]) * dinv + b1_ref[...]
    h1 = jnp.maximum(pre, 0.0)
    s2_ref[...] = jnp.dot(h1, w2_ref[...],
                          preferred_element_type=jnp.float32) * dinv


_tc2 = pl.pallas_call(
    _tc2_body,
    grid=(N_PAD // BR,),
    in_specs=[
        pl.BlockSpec((BR, 64), lambda i: (i, 0)),
        pl.BlockSpec((NC, BR, 64), lambda i: (0, i, 0)),
        pl.BlockSpec((BR, 1), lambda i: (i, 0)),
        pl.BlockSpec((1, 64), lambda i: (0, 0)),
        pl.BlockSpec((64, 32), lambda i: (0, 0)),
    ],
    out_specs=pl.BlockSpec((BR, 32), lambda i: (i, 0)),
    out_shape=jax.ShapeDtypeStruct((N_PAD, 32), jnp.float32),
)


def _tc3_body(s2_ref, aggp_ref, dinv_ref, b2_ref, wg_ref, bg_ref,
              wh_ref, bh_ref, y_ref):
    h2 = (aggp_ref[0] + aggp_ref[1] + s2_ref[...]) * dinv_ref[...] + b2_ref[...]
    gate = jnp.dot(h2, wg_ref[...], preferred_element_type=jnp.float32)
    gate = gate + bg_ref[...]
    gate = 1.0 / (1.0 + jnp.exp(-gate))
    valid = lax.broadcasted_iota(jnp.int32, (N_PAD, 1), 0) < N
    g = jnp.where(valid, gate, -1e30)
    m = jnp.max(g)
    e = jnp.where(valid, jnp.exp(g - m), 0.0)
    hg = jnp.sum(e * h2, axis=0, keepdims=True) / jnp.sum(e)
    y_ref[...] = jnp.dot(hg, wh_ref[...],
                         preferred_element_type=jnp.float32) + bh_ref[...]


_tc3 = pl.pallas_call(
    _tc3_body,
    in_specs=[
        pl.BlockSpec((N_PAD, 32), lambda: (0, 0)),
        pl.BlockSpec((NC, N_PAD, 32), lambda: (0, 0, 0)),
        pl.BlockSpec((N_PAD, 1), lambda: (0, 0)),
        pl.BlockSpec((1, 32), lambda: (0, 0)),
        pl.BlockSpec((32, 1), lambda: (0, 0)),
        pl.BlockSpec((1, 1), lambda: (0, 0)),
        pl.BlockSpec((32, 2), lambda: (0, 0)),
        pl.BlockSpec((1, 2), lambda: (0, 0)),
    ],
    out_specs=pl.BlockSpec((1, 2), lambda: (0, 0)),
    out_shape=jax.ShapeDtypeStruct((1, 2), jnp.float32),
)


def kernel(x, edge_index, edge_attr, embed_table, edge_embed_table,
           W1, b1, W2, b2, Wg, bg, Wh, bh):
    del edge_attr, edge_embed_table
    xi = x[:, 0].astype(jnp.int32)
    x_pad = jnp.concatenate(
        [xi, jnp.zeros((N_PAD - N,), jnp.int32)]).reshape(NW, X_CHUNKS, XCH)
    row = edge_index[0].astype(jnp.int32)
    col = edge_index[1].astype(jnp.int32)
    dump = N + (jnp.arange(E_PAD - E, dtype=jnp.int32) % (N_PAD - N))
    row_flat = jnp.concatenate([row, jnp.zeros((E_PAD - E,), jnp.int32)])
    col_flat = jnp.concatenate([col, dump])
    row_ch = row_flat.reshape(NW, E_CHUNKS, ECH)
    col_ch = col_flat.reshape(NW, E_CHUNKS, ECH)
    col_pad = col_ch
    zcol = jnp.zeros((N_PAD, 16), jnp.float32)
    ones = jnp.ones((ECH, 16), jnp.float32)
    z64 = jnp.zeros((N_PAD, 64), jnp.float32)
    z32 = jnp.zeros((N_PAD, 32), jnp.float32)

    h0 = _sc_embed(x_pad, embed_table)
    degp = _sc_deg(col_pad, zcol, ones)
    s1, dinv = _tc1(h0, degp, W1)
    aggp1 = _agg64(row_ch, col_ch, s1, z64)
    s2 = _tc2(s1, aggp1, dinv, b1.reshape(1, 64), W2)
    aggp2 = _agg32(row_ch, col_ch, s2, z32)
    y = _tc3(s2, aggp2, dinv, b2.reshape(1, 32), Wg, bg.reshape(1, 1),
             Wh, bh.reshape(1, 2))
    return y


# speedup vs baseline: 1.9470x; 1.8356x over previous
import functools

import jax
import jax.numpy as jnp
from jax import lax
from jax.experimental import pallas as pl
from jax.experimental.pallas import tpu as pltpu
from jax.experimental.pallas import tpu_sc as plsc

N = 10000
E = 160000
D = 256
NC = 2
NS = 16
NW = NC * NS
N_PAD = 10240
E_PAD = 163840
ECH = 512
E_PER_W = E_PAD // NW
E_CHUNKS = E_PER_W // ECH
XCH = 64
X_PER_W = N_PAD // NW
X_CHUNKS = X_PER_W // XCH
RPS = N_PAD // NS


def _mesh():
    return plsc.VectorSubcoreMesh(core_axis_name="c", subcore_axis_name="s")


@functools.partial(
    pl.kernel,
    out_type=jax.ShapeDtypeStruct((N_PAD, D), jnp.float32),
    mesh=_mesh(),
    scratch_types=[
        pltpu.VMEM((X_CHUNKS, XCH), jnp.int32),
        pltpu.VMEM((XCH, D), jnp.float32),
        pltpu.VMEM((XCH, D), jnp.float32),
        pltpu.SemaphoreType.DMA,
        pltpu.SemaphoreType.DMA,
    ],
    name="sc_embed",
)
def _sc_embed(x2d, table, h0, xidx, rows_a, rows_b, sem_a, sem_b):
    cid = lax.axis_index("c")
    sid = lax.axis_index("s")
    wid = sid * NC + cid
    pltpu.sync_copy(x2d.at[wid], xidx)
    bufs = (rows_a, rows_b)
    sems = (sem_a, sem_b)
    descs = [None, None]
    for j in range(X_CHUNKS):
        b = j % 2
        if descs[b] is not None:
            descs[b].wait()
            pltpu.sync_copy(bufs[b],
                            h0.at[pl.ds(wid * X_PER_W + (j - 2) * XCH, XCH)])
        descs[b] = pltpu.async_copy(table.at[xidx.at[j]], bufs[b], sems[b])
    for j in range(X_CHUNKS - 2, X_CHUNKS):
        b = j % 2
        descs[b].wait()
        pltpu.sync_copy(bufs[b],
                        h0.at[pl.ds(wid * X_PER_W + j * XCH, XCH)])


@functools.partial(
    pl.kernel,
    out_type=jax.ShapeDtypeStruct((NC, N_PAD, 16), jnp.float32),
    mesh=_mesh(),
    scratch_types=[
        pltpu.VMEM((E_CHUNKS, ECH), jnp.int32),
        pltpu.VMEM((ECH, 16), jnp.float32),
        pltpu.VMEM_SHARED((N_PAD, 16), jnp.float32),
        pltpu.SemaphoreType.DMA,
    ],
    compiler_params=pltpu.CompilerParams(use_tc_tiling_on_sc=False),
    name="sc_deg",
)
def _sc_deg(col2d, zcol, ones, degp, cidx, ones_v, deg_sh, sem):
    cid = lax.axis_index("c")
    sid = lax.axis_index("s")
    wid = sid * NC + cid
    pltpu.sync_copy(zcol.at[pl.ds(sid * RPS, RPS)],
                    deg_sh.at[pl.ds(sid * RPS, RPS)])
    pltpu.sync_copy(ones, ones_v)
    pltpu.sync_copy(col2d.at[wid], cidx)
    plsc.subcore_barrier()

    for k in range(E_CHUNKS):
        pltpu.async_copy(ones_v, deg_sh.at[cidx.at[k]], sem, add=True)
    for _ in range(E_CHUNKS):
        pltpu.make_async_copy(ones_v, deg_sh.at[cidx.at[0]], sem).wait()
    plsc.subcore_barrier()
    pltpu.sync_copy(deg_sh.at[pl.ds(sid * RPS, RPS)],
                    degp.at[cid, pl.ds(sid * RPS, RPS)])


def _make_agg(dm):
    @functools.partial(
        pl.kernel,
        out_type=jax.ShapeDtypeStruct((NC, N_PAD, dm), jnp.float32),
        mesh=_mesh(),
        scratch_types=[
            pltpu.VMEM((E_CHUNKS, ECH), jnp.int32),
            pltpu.VMEM((E_CHUNKS, ECH), jnp.int32),
            [pltpu.VMEM((ECH, dm), jnp.float32)] * 2,
            pltpu.VMEM_SHARED((N_PAD, dm), jnp.float32),
            [pltpu.SemaphoreType.DMA] * 2,
        ],
        compiler_params=pltpu.CompilerParams(use_tc_tiling_on_sc=False),
        name=f"sc_agg{dm}",
    )
    def agg(row2d, col2d, s_hbm, zeros_hbm, aggp,
            ridx, cidx, msgs, agg_sh, sem_g):
        cid = lax.axis_index("c")
        sid = lax.axis_index("s")
        wid = sid * NC + cid
        nch = E_CHUNKS
        pltpu.sync_copy(zeros_hbm.at[pl.ds(sid * RPS, RPS)],
                        agg_sh.at[pl.ds(sid * RPS, RPS)])
        pltpu.sync_copy(row2d.at[wid], ridx)
        pltpu.sync_copy(col2d.at[wid], cidx)
        plsc.subcore_barrier()

        pltpu.async_copy(s_hbm.at[ridx.at[0]], msgs[0], sem_g[0])

        def pair(j2, c):
            ja = 2 * j2
            jb = ja + 1
            ja_next = jnp.minimum(ja + 2, nch - 1)
            pltpu.async_copy(s_hbm.at[ridx.at[jb]], msgs[1], sem_g[1])
            pltpu.make_async_copy(s_hbm.at[ridx.at[0]], msgs[0],
                                  sem_g[0]).wait()
            pltpu.sync_copy(msgs[0], agg_sh.at[cidx.at[ja]], add=True)
            pltpu.async_copy(s_hbm.at[ridx.at[ja_next]], msgs[0], sem_g[0])
            pltpu.make_async_copy(s_hbm.at[ridx.at[0]], msgs[1],
                                  sem_g[1]).wait()
            pltpu.sync_copy(msgs[1], agg_sh.at[cidx.at[jb]], add=True)
            return c

        lax.fori_loop(0, nch // 2, pair, 0)
        pltpu.make_async_copy(s_hbm.at[ridx.at[0]], msgs[0], sem_g[0]).wait()
        plsc.subcore_barrier()
        pltpu.sync_copy(agg_sh.at[pl.ds(sid * RPS, RPS)],
                        aggp.at[cid, pl.ds(sid * RPS, RPS)])

    return agg


_agg64 = _make_agg(64)
_agg32 = _make_agg(32)


BR = 1024


def _tc1_body(h0_ref, degp_ref, w1_ref, s1_ref, dinv_ref):
    deg = degp_ref[0, :, 0:1] + degp_ref[1, :, 0:1] + 1.0
    dinv = lax.rsqrt(deg)
    xw = jnp.dot(h0_ref[...], w1_ref[...], preferred_element_type=jnp.float32)
    s1_ref[...] = xw * dinv
    dinv_ref[...] = dinv


_tc1 = pl.pallas_call(
    _tc1_body,
    grid=(N_PAD // BR,),
    in_specs=[
        pl.BlockSpec((BR, D), lambda i: (i, 0)),
        pl.BlockSpec((NC, BR, 16), lambda i: (0, i, 0)),
        pl.BlockSpec((D, 64), lambda i: (0, 0)),
    ],
    out_specs=[
        pl.BlockSpec((BR, 64), lambda i: (i, 0)),
        pl.BlockSpec((BR, 1), lambda i: (i, 0)),
    ],
    out_shape=[
        jax.ShapeDtypeStruct((N_PAD, 64), jnp.float32),
        jax.ShapeDtypeStruct((N_PAD, 1), jnp.float32),
    ],
)


def _tc2_body(s1_ref, aggp_ref, dinv_ref, b1_ref, w2_ref, s2_ref):
    dinv = dinv_ref[...]
    pre = (aggp_ref[0] + aggp_ref[1] + s1_ref[...]) * dinv + b1_ref[...]
    h1 = jnp.maximum(pre, 0.0)
    s2_ref[...] = jnp.dot(h1, w2_ref[...],
                          preferred_element_type=jnp.float32) * dinv


_tc2 = pl.pallas_call(
    _tc2_body,
    grid=(N_PAD // BR,),
    in_specs=[
        pl.BlockSpec((BR, 64), lambda i: (i, 0)),
        pl.BlockSpec((NC, BR, 64), lambda i: (0, i, 0)),
        pl.BlockSpec((BR, 1), lambda i: (i, 0)),
        pl.BlockSpec((1, 64), lambda i: (0, 0)),
        pl.BlockSpec((64, 32), lambda i: (0, 0)),
    ],
    out_specs=pl.BlockSpec((BR, 32), lambda i: (i, 0)),
    out_shape=jax.ShapeDtypeStruct((N_PAD, 32), jnp.float32),
)


def _tc3_body(s2_ref, aggp_ref, dinv_ref, b2_ref, wg_ref, bg_ref,
              wh_ref, bh_ref, y_ref):
    h2 = (aggp_ref[0] + aggp_ref[1] + s2_ref[...]) * dinv_ref[...] + b2_ref[...]
    gate = jnp.dot(h2, wg_ref[...], preferred_element_type=jnp.float32)
    gate = gate + bg_ref[...]
    gate = 1.0 / (1.0 + jnp.exp(-gate))
    valid = lax.broadcasted_iota(jnp.int32, (N_PAD, 1), 0) < N
    g = jnp.where(valid, gate, -1e30)
    m = jnp.max(g)
    e = jnp.where(valid, jnp.exp(g - m), 0.0)
    hg = jnp.sum(e * h2, axis=0, keepdims=True) / jnp.sum(e)
    y_ref[...] = jnp.dot(hg, wh_ref[...],
                         preferred_element_type=jnp.float32) + bh_ref[...]


_tc3 = pl.pallas_call(
    _tc3_body,
    in_specs=[
        pl.BlockSpec((N_PAD, 32), lambda: (0, 0)),
        pl.BlockSpec((NC, N_PAD, 32), lambda: (0, 0, 0)),
        pl.BlockSpec((N_PAD, 1), lambda: (0, 0)),
        pl.BlockSpec((1, 32), lambda: (0, 0)),
        pl.BlockSpec((32, 1), lambda: (0, 0)),
        pl.BlockSpec((1, 1), lambda: (0, 0)),
        pl.BlockSpec((32, 2), lambda: (0, 0)),
        pl.BlockSpec((1, 2), lambda: (0, 0)),
    ],
    out_specs=pl.BlockSpec((1, 2), lambda: (0, 0)),
    out_shape=jax.ShapeDtypeStruct((1, 2), jnp.float32),
)


def kernel(x, edge_index, edge_attr, embed_table, edge_embed_table,
           W1, b1, W2, b2, Wg, bg, Wh, bh):
    del edge_attr, edge_embed_table
    xi = x[:, 0].astype(jnp.int32)
    x_pad = jnp.concatenate(
        [xi, jnp.arange(N_PAD - N, dtype=jnp.int32)]).reshape(
            NW, X_CHUNKS, XCH)
    row = edge_index[0].astype(jnp.int32)
    col = edge_index[1].astype(jnp.int32)
    dump = N + (jnp.arange(E_PAD - E, dtype=jnp.int32) % (N_PAD - N))
    spread = jnp.arange(E_PAD - E, dtype=jnp.int32) % N
    row_flat = jnp.concatenate([row, spread])
    col_flat = jnp.concatenate([col, dump])
    row_ch = row_flat.reshape(NW, E_CHUNKS, ECH)
    col_ch = col_flat.reshape(NW, E_CHUNKS, ECH)
    col_pad = col_ch
    zcol = jnp.zeros((N_PAD, 16), jnp.float32)
    ones = jnp.ones((ECH, 16), jnp.float32)
    z64 = jnp.zeros((N_PAD, 64), jnp.float32)
    z32 = jnp.zeros((N_PAD, 32), jnp.float32)

    h0 = _sc_embed(x_pad, embed_table)
    degp = _sc_deg(col_pad, zcol, ones)
    s1, dinv = _tc1(h0, degp, W1)
    aggp1 = _agg64(row_ch, col_ch, s1, z64)
    s2 = _tc2(s1, aggp1, dinv, b1.reshape(1, 64), W2)
    aggp2 = _agg32(row_ch, col_ch, s2, z32)
    y = _tc3(s2, aggp2, dinv, b2.reshape(1, 32), Wg, bg.reshape(1, 1),
             Wh, bh.reshape(1, 2))
    return y
